# Initial kernel scaffold; baseline (speedup 1.0000x reference)
import jax
import jax.numpy as jnp
from jax.experimental import pallas as pl


def kernel(hidden_states, attention_mask, heads, rels, W_self, W_head, b):
    raise NotImplementedError("write your pallas kernel here")


# speedup vs baseline: 1999.8778x; 1999.8778x over previous
import functools

import jax
import jax.numpy as jnp
from jax import lax
from jax.experimental import pallas as pl
from jax.experimental.pallas import tpu as pltpu
from jax.experimental.pallas import tpu_sc as plsc


def _gather_rows(x2, gidx):
    rows, hdim = x2.shape
    info = plsc.get_sparse_core_info()
    ncores, nsub = info.num_cores, info.num_subcores
    nw = ncores * nsub
    rows_per_w = rows // nw
    chunk = min(128, rows_per_w)
    n_chunks = rows_per_w // chunk
    mesh = plsc.VectorSubcoreMesh(core_axis_name="c", subcore_axis_name="s")

    @functools.partial(
        pl.kernel,
        mesh=mesh,
        out_type=jax.ShapeDtypeStruct((rows, hdim), jnp.float32),
        scratch_types=[
            pltpu.VMEM((chunk,), jnp.int32),
            pltpu.VMEM((chunk, hdim), jnp.float32),
            pltpu.SemaphoreType.DMA,
        ],
    )
    def gk(x_hbm, idx_hbm, out_hbm, idx_v, rows_v, sem):
        wid = lax.axis_index("s") * ncores + lax.axis_index("c")
        for c in range(n_chunks):
            base = wid * rows_per_w + c * chunk
            pltpu.sync_copy(idx_hbm.at[pl.ds(base, chunk)], idx_v)
            pltpu.async_copy(x_hbm.at[idx_v], rows_v, sem).wait()
            pltpu.sync_copy(rows_v, out_hbm.at[pl.ds(base, chunk)])

    return gk(x2, gidx)


def _layer(x2, h2, w_self, w_head, bias, mask2):
    rows, hdim = x2.shape
    bm = 256
    grid = (rows // bm,)

    def body(x_ref, h_ref, ws_ref, wh_ref, b_ref, m_ref, o_ref):
        acc = jnp.dot(x_ref[...], ws_ref[...], preferred_element_type=jnp.float32)
        acc = acc + jnp.dot(h_ref[...], wh_ref[...], preferred_element_type=jnp.float32)
        acc = acc + b_ref[...]
        o_ref[...] = jnp.maximum(acc, 0.0) * m_ref[...]

    return pl.pallas_call(
        body,
        grid=grid,
        in_specs=[
            pl.BlockSpec((bm, hdim), lambda i: (i, 0)),
            pl.BlockSpec((bm, hdim), lambda i: (i, 0)),
            pl.BlockSpec((hdim, hdim), lambda i: (0, 0)),
            pl.BlockSpec((hdim, hdim), lambda i: (0, 0)),
            pl.BlockSpec((1, hdim), lambda i: (0, 0)),
            pl.BlockSpec((bm, 1), lambda i: (i, 0)),
        ],
        out_specs=pl.BlockSpec((bm, hdim), lambda i: (i, 0)),
        out_shape=jax.ShapeDtypeStruct((rows, hdim), jnp.float32),
    )(x2, h2, w_self, w_head, bias, mask2)


def kernel(hidden_states, attention_mask, heads, rels, W_self, W_head, b):
    del rels
    bsz, seq, hdim = hidden_states.shape
    rows = bsz * seq
    x2 = hidden_states.reshape(rows, hdim)
    offs = (jnp.arange(bsz, dtype=jnp.int32) * seq)[:, None]
    gidx = (heads.astype(jnp.int32) + offs).reshape(rows)
    mask2 = attention_mask.reshape(rows, 1)
    num_layers = W_self.shape[0]
    for l in range(num_layers):
        h2 = _gather_rows(x2, gidx)
        x2 = _layer(x2, h2, W_self[l], W_head[l], b[l].reshape(1, hdim), mask2)
    return x2.reshape(bsz, seq, hdim)
